# baseline (speedup 1.0000x reference)
import functools

import jax
import jax.numpy as jnp
from jax import lax
from jax.experimental import pallas as pl
from jax.experimental.pallas import tpu as pltpu
from jax.experimental.pallas import tpu_sc as plsc


def _vq_codes_body(x_ref, cb_ref, codes_ref, loss_ref):
    xb = x_ref[0]
    cb = cb_ref[...]

    x_sq = jnp.sum(xb * xb, axis=0)
    c_sq = jnp.sum(cb * cb, axis=1)

    mm = lax.dot_general(cb, xb, (((1,), (0,)), ((), ())),
                         preferred_element_type=jnp.float32)
    d = (x_sq[None, :] - 2.0 * mm) + c_sq[:, None]

    mins = jnp.min(d, axis=0, keepdims=True)
    iota_c = lax.broadcasted_iota(jnp.int32, d.shape, 0)
    cand = jnp.where(d == mins, iota_c, jnp.int32(2 ** 30))
    codes = jnp.min(cand, axis=0)

    codes_ref[...] = codes.reshape(1, 1, codes.shape[0])
    loss_ref[...] = jnp.broadcast_to(jnp.sum(mins), (1, 8, 128))


def _tc_codes(x, codebook):
    batch, dim, frames = x.shape
    ncodes = codebook.shape[0]
    codes3, lossp = pl.pallas_call(
        _vq_codes_body,
        grid=(batch,),
        in_specs=[
            pl.BlockSpec((1, dim, frames), lambda b: (b, 0, 0)),
            pl.BlockSpec((ncodes, dim), lambda b: (0, 0)),
        ],
        out_specs=[
            pl.BlockSpec((1, 1, frames), lambda b: (b, 0, 0)),
            pl.BlockSpec((1, 8, 128), lambda b: (b, 0, 0)),
        ],
        out_shape=[
            jax.ShapeDtypeStruct((batch, 1, frames), jnp.int32),
            jax.ShapeDtypeStruct((batch, 8, 128), jnp.float32),
        ],
    )(x, codebook)
    return codes3.reshape(batch, frames), lossp


def _sc_gather(cbT_flat, codes_flat, batch, dim, frames):
    NC, NS, L = 2, 16, 16
    NW = NC * NS
    DG = dim // NW
    CH = frames // L
    mesh = plsc.VectorSubcoreMesh(core_axis_name="c", subcore_axis_name="s")

    @functools.partial(
        pl.kernel, mesh=mesh,
        out_type=jax.ShapeDtypeStruct((batch * dim * frames,), jnp.float32),
        scratch_types=[
            pltpu.VMEM((DG * frames,), jnp.float32),
            pltpu.VMEM((frames,), jnp.int32),
            pltpu.VMEM((DG * frames,), jnp.float32),
        ],
        compiler_params=pltpu.CompilerParams(needs_layout_passes=False),
    )
    def k(cbT_hbm, codes_hbm, out_hbm, cbt_v, codes_v, rows_v):
        wid = lax.axis_index("s") * NC + lax.axis_index("c")
        dbase = wid * DG
        pltpu.sync_copy(cbT_hbm.at[pl.ds(dbase * frames, DG * frames)], cbt_v)

        def per_batch(b, carry):
            pltpu.sync_copy(codes_hbm.at[pl.ds(b * frames, frames)], codes_v)
            for d in range(DG):
                off = jnp.int32(d * frames)

                def chunk(j, c):
                    idx = codes_v[pl.ds(j * L, L)]
                    vals = plsc.load_gather(cbt_v, [idx + off])
                    rows_v[pl.ds(d * frames + j * L, L)] = vals
                    return c

                lax.fori_loop(0, CH, chunk, 0, unroll=4)
            pltpu.sync_copy(
                rows_v,
                out_hbm.at[pl.ds(b * (dim * frames) + dbase * frames,
                                 DG * frames)])
            return carry

        lax.fori_loop(0, batch, per_batch, 0)

    return k(cbT_flat, codes_flat).reshape(batch, dim, frames)


def kernel(x, codebook):
    batch, dim, frames = x.shape
    codes, lossp = _tc_codes(x, codebook)
    cbT_flat = codebook.T.reshape(-1)
    q = _sc_gather(cbT_flat, codes.reshape(-1), batch, dim, frames)
    vq_loss = 1.25 * jnp.sum(lossp[:, 0, 0]) / (batch * dim * frames)
    return (q, codes, vq_loss)


# speedup vs baseline: 1.3313x; 1.3313x over previous
import functools

import jax
import jax.numpy as jnp
from jax import lax
from jax.experimental import pallas as pl
from jax.experimental.pallas import tpu as pltpu
from jax.experimental.pallas import tpu_sc as plsc


def _vq_codes_body(x_ref, cb_ref, codes_ref, loss_ref):
    xb = x_ref[0]
    cb = cb_ref[...]

    x_sq = jnp.sum(xb * xb, axis=0)
    c_sq = jnp.sum(cb * cb, axis=1)

    mm = lax.dot_general(cb, xb, (((1,), (0,)), ((), ())),
                         preferred_element_type=jnp.float32)
    d = (x_sq[None, :] - 2.0 * mm) + c_sq[:, None]

    mins = jnp.min(d, axis=0, keepdims=True)
    iota_c = lax.broadcasted_iota(jnp.int32, d.shape, 0)
    cand = jnp.where(d == mins, iota_c, jnp.int32(2 ** 30))
    codes = jnp.min(cand, axis=0)

    codes_ref[...] = codes.reshape(1, 1, codes.shape[0])
    loss_ref[...] = jnp.broadcast_to(jnp.sum(mins), (1, 8, 128))


def _tc_codes(x, codebook):
    batch, dim, frames = x.shape
    ncodes = codebook.shape[0]
    codes3, lossp = pl.pallas_call(
        _vq_codes_body,
        grid=(batch,),
        in_specs=[
            pl.BlockSpec((1, dim, frames), lambda b: (b, 0, 0)),
            pl.BlockSpec((ncodes, dim), lambda b: (0, 0)),
        ],
        out_specs=[
            pl.BlockSpec((1, 1, frames), lambda b: (b, 0, 0)),
            pl.BlockSpec((1, 8, 128), lambda b: (b, 0, 0)),
        ],
        out_shape=[
            jax.ShapeDtypeStruct((batch, 1, frames), jnp.int32),
            jax.ShapeDtypeStruct((batch, 8, 128), jnp.float32),
        ],
    )(x, codebook)
    return codes3.reshape(batch, frames), lossp


def _sc_gather(cbT_flat, codes_flat, batch, dim, frames):
    NC, NS, L = 2, 16, 16
    NW = NC * NS
    DG = dim // NW
    CH = frames // L
    mesh = plsc.VectorSubcoreMesh(core_axis_name="c", subcore_axis_name="s")

    @functools.partial(
        pl.kernel, mesh=mesh,
        out_type=jax.ShapeDtypeStruct((batch * dim * frames,), jnp.float32),
        scratch_types=[
            pltpu.VMEM((DG * frames,), jnp.float32),
            pltpu.VMEM((frames,), jnp.int32),
            pltpu.VMEM((DG * frames,), jnp.float32),
        ],
        compiler_params=pltpu.CompilerParams(needs_layout_passes=False),
    )
    def k(cbT_hbm, codes_hbm, out_hbm, cbt_v, codes_v, rows_v):
        wid = lax.axis_index("s") * NC + lax.axis_index("c")
        dbase = wid * DG
        pltpu.sync_copy(cbT_hbm.at[pl.ds(dbase * frames, DG * frames)], cbt_v)

        def per_batch(b, carry):
            pltpu.sync_copy(codes_hbm.at[pl.ds(b * frames, frames)], codes_v)

            def chunk(j, c):
                idx = codes_v[pl.ds(j * L, L)]
                for d in range(DG):
                    vals = plsc.load_gather(cbt_v, [idx + jnp.int32(d * frames)])
                    rows_v[pl.ds(d * frames + j * L, L)] = vals
                return c

            lax.fori_loop(0, CH, chunk, 0, unroll=8)
            pltpu.sync_copy(
                rows_v,
                out_hbm.at[pl.ds(b * (dim * frames) + dbase * frames,
                                 DG * frames)])
            return carry

        lax.fori_loop(0, batch, per_batch, 0)

    return k(cbT_flat, codes_flat).reshape(batch, dim, frames)


def kernel(x, codebook):
    batch, dim, frames = x.shape
    codes, lossp = _tc_codes(x, codebook)
    cbT_flat = codebook.T.reshape(-1)
    q = _sc_gather(cbT_flat, codes.reshape(-1), batch, dim, frames)
    vq_loss = 1.25 * jnp.sum(lossp[:, 0, 0]) / (batch * dim * frames)
    return (q, codes, vq_loss)


# speedup vs baseline: 2.9459x; 2.2127x over previous
import jax
import jax.numpy as jnp
from jax import lax
from jax.experimental import pallas as pl
from jax.experimental.pallas import tpu as pltpu

_FB = 1024


def _vq_body(x_ref, cb_ref, q_ref, codes_ref, loss_ref, cbcat_ref):
    xb = x_ref[0]
    cb = cb_ref[...]

    @pl.when(jnp.logical_and(pl.program_id(0) == 0, pl.program_id(1) == 0))
    def _init():
        bits = lax.bitcast_convert_type(cb, jnp.uint32)
        hi32 = lax.bitcast_convert_type(bits & jnp.uint32(0xFFFF0000),
                                        jnp.float32)
        hi_bf = hi32.astype(jnp.bfloat16)
        lo_bf = ((cb - hi32) * 512.0).astype(jnp.bfloat16)
        cbcat_ref[...] = jnp.concatenate([hi_bf, lo_bf], axis=1)

    x_sq = jnp.sum(xb * xb, axis=0)
    c_sq = jnp.sum(cb * cb, axis=1)

    mm = lax.dot_general(cb, xb, (((1,), (0,)), ((), ())),
                         preferred_element_type=jnp.float32)
    d = (x_sq[None, :] - 2.0 * mm) + c_sq[:, None]

    mins = jnp.min(d, axis=0, keepdims=True)
    iota_c = lax.broadcasted_iota(jnp.int32, d.shape, 0)
    cand = jnp.where(d == mins, iota_c, jnp.int32(2 ** 30))
    codes = jnp.min(cand, axis=0)

    onehot = (cand == codes[None, :]).astype(jnp.bfloat16)
    dn = (((0,), (0,)), ((), ()))
    qq = lax.dot_general(cbcat_ref[...], onehot, dn,
                         preferred_element_type=jnp.float32)
    dim = qq.shape[0] // 2
    q = qq[:dim] + qq[dim:] * (1.0 / 512.0)

    q_ref[0] = q
    codes_ref[...] = codes.reshape(1, 1, codes.shape[0])
    loss_ref[...] = jnp.broadcast_to(jnp.sum(mins), (1, 1, 8, 128))


def kernel(x, codebook):
    batch, dim, frames = x.shape
    ncodes = codebook.shape[0]
    nf = frames // _FB

    q, codes3, lossp = pl.pallas_call(
        _vq_body,
        grid=(batch, nf),
        in_specs=[
            pl.BlockSpec((1, dim, _FB), lambda b, f: (b, 0, f)),
            pl.BlockSpec((ncodes, dim), lambda b, f: (0, 0)),
        ],
        out_specs=[
            pl.BlockSpec((1, dim, _FB), lambda b, f: (b, 0, f)),
            pl.BlockSpec((1, 1, _FB), lambda b, f: (b, 0, f)),
            pl.BlockSpec((1, 1, 8, 128), lambda b, f: (b, f, 0, 0)),
        ],
        out_shape=[
            jax.ShapeDtypeStruct((batch, dim, frames), jnp.float32),
            jax.ShapeDtypeStruct((batch, 1, frames), jnp.int32),
            jax.ShapeDtypeStruct((batch, nf, 8, 128), jnp.float32),
        ],
        scratch_shapes=[pltpu.VMEM((ncodes, 2 * dim), jnp.bfloat16)],
        compiler_params=pltpu.CompilerParams(
            dimension_semantics=("parallel", "parallel"),
        ),
    )(x, codebook)

    codes = codes3.reshape(batch, frames)
    vq_loss = 1.25 * jnp.sum(lossp[:, :, 0, 0]) / (batch * dim * frames)
    return (q, codes, vq_loss)


# speedup vs baseline: 2.9502x; 1.0015x over previous
import jax
import jax.numpy as jnp
from jax import lax
from jax.experimental import pallas as pl
from jax.experimental.pallas import tpu as pltpu

_FB = 1024


def _vq_body(x_ref, cb_ref, q_ref, codes_ref, loss_ref, cbcat_ref):
    xb = x_ref[0]
    cb = cb_ref[...]

    @pl.when(jnp.logical_and(pl.program_id(0) == 0, pl.program_id(1) == 0))
    def _init():
        bits = lax.bitcast_convert_type(cb, jnp.uint32)
        hi32 = lax.bitcast_convert_type(bits & jnp.uint32(0xFFFF0000),
                                        jnp.float32)
        hi_bf = hi32.astype(jnp.bfloat16)
        lo_bf = ((cb - hi32) * 512.0).astype(jnp.bfloat16)
        cbcat_ref[...] = jnp.concatenate([hi_bf, lo_bf], axis=1)

    x_sq = jnp.sum(xb * xb, axis=0)
    c_sq = jnp.sum(cb * cb, axis=1)

    mm = lax.dot_general(cb, xb, (((1,), (0,)), ((), ())),
                         preferred_element_type=jnp.float32)
    d = (x_sq[None, :] - 2.0 * mm) + c_sq[:, None]

    mins = jnp.min(d, axis=0, keepdims=True)
    iota_c = lax.broadcasted_iota(jnp.int32, d.shape, 0)
    cand = jnp.where(d == mins, iota_c, jnp.int32(2 ** 30))
    codes = jnp.min(cand, axis=0)

    onehot = (cand == codes[None, :]).astype(jnp.bfloat16)
    dn = (((0,), (0,)), ((), ()))
    qq = lax.dot_general(cbcat_ref[...], onehot, dn,
                         preferred_element_type=jnp.float32)
    dim = qq.shape[0] // 2
    q = qq[:dim] + qq[dim:] * (1.0 / 512.0)

    q_ref[0] = q
    codes_ref[...] = codes.reshape(1, 1, codes.shape[0])
    loss_ref[...] = jnp.broadcast_to(jnp.sum(mins), (1, 1, 8, 128))


def kernel(x, codebook):
    batch, dim, frames = x.shape
    ncodes = codebook.shape[0]
    nf = frames // _FB

    q, codes3, lossp = pl.pallas_call(
        _vq_body,
        grid=(batch, nf),
        in_specs=[
            pl.BlockSpec((1, dim, _FB), lambda b, f: (b, 0, f)),
            pl.BlockSpec((ncodes, dim), lambda b, f: (0, 0)),
        ],
        out_specs=[
            pl.BlockSpec((1, dim, _FB), lambda b, f: (b, 0, f)),
            pl.BlockSpec((1, 1, _FB), lambda b, f: (b, 0, f)),
            pl.BlockSpec((1, 1, 8, 128), lambda b, f: (b, f, 0, 0)),
        ],
        out_shape=[
            jax.ShapeDtypeStruct((batch, dim, frames), jnp.float32),
            jax.ShapeDtypeStruct((batch, 1, frames), jnp.int32),
            jax.ShapeDtypeStruct((batch, nf, 8, 128), jnp.float32),
        ],
        scratch_shapes=[pltpu.VMEM((ncodes, 2 * dim), jnp.bfloat16)],
        compiler_params=pltpu.CompilerParams(
            dimension_semantics=("arbitrary", "arbitrary"),
        ),
    )(x, codebook)

    codes = codes3.reshape(batch, frames)
    vq_loss = 1.25 * jnp.sum(lossp[:, :, 0, 0]) / (batch * dim * frames)
    return (q, codes, vq_loss)
